# Initial kernel scaffold; baseline (speedup 1.0000x reference)
import jax
import jax.numpy as jnp
from jax.experimental import pallas as pl


def kernel(edge_attr, W0, W1, W2):
    raise NotImplementedError("write your pallas kernel here")


# speedup vs baseline: 2.1331x; 2.1331x over previous
import functools

import jax
import jax.numpy as jnp
from jax import lax
from jax.experimental import pallas as pl
from jax.experimental.pallas import tpu as pltpu
from jax.experimental.pallas import tpu_sc as plsc

_D0, _D1, _D2 = 6, 7, 3
_EMB = 128
_TROWS = 128
_EDGE_BLK = 3200

_NC, _NS = 2, 16
_NW = _NC * _NS
_CHUNK = 128


def _tc_prep_body(ea_ref, w0_ref, w1_ref, w2_ref, t_ref, cidx_ref):
    e = ea_ref[...]
    e0 = jnp.clip(e[:, 0], 0, _D0 - 1)
    e1 = jnp.clip(e[:, 1], 0, _D1 - 1)
    e2 = jnp.clip(e[:, 2], 0, _D2 - 1)
    cidx = e0 * (_D1 * _D2) + e1 * _D2 + e2
    cidx_ref[...] = cidx.reshape(1, _EDGE_BLK // _EMB, _EMB)

    @pl.when(pl.program_id(0) == 0)
    def _build_table():
        r = lax.broadcasted_iota(jnp.int32, (_TROWS, 1), 0)
        i0 = r // (_D1 * _D2)
        i1 = (r // _D2) % _D1
        i2 = r % _D2
        oh0 = (i0 == lax.broadcasted_iota(jnp.int32, (_TROWS, _D0), 1)).astype(jnp.float32)
        oh1 = (i1 == lax.broadcasted_iota(jnp.int32, (_TROWS, _D1), 1)).astype(jnp.float32)
        oh2 = (i2 == lax.broadcasted_iota(jnp.int32, (_TROWS, _D2), 1)).astype(jnp.float32)
        hi = lax.Precision.HIGHEST
        t_ref[...] = (
            jnp.dot(oh0, w0_ref[...], precision=hi, preferred_element_type=jnp.float32)
            + jnp.dot(oh1, w1_ref[...], precision=hi, preferred_element_type=jnp.float32)
            + jnp.dot(oh2, w2_ref[...], precision=hi, preferred_element_type=jnp.float32)
        )


def _tc_prep(edge_attr, W0, W1, W2, interpret=False):
    n = edge_attr.shape[0]
    return pl.pallas_call(
        _tc_prep_body,
        grid=(n // _EDGE_BLK,),
        in_specs=[
            pl.BlockSpec((_EDGE_BLK, 3), lambda i: (i, 0)),
            pl.BlockSpec((_D0, _EMB), lambda i: (0, 0)),
            pl.BlockSpec((_D1, _EMB), lambda i: (0, 0)),
            pl.BlockSpec((_D2, _EMB), lambda i: (0, 0)),
        ],
        out_specs=[
            pl.BlockSpec((_TROWS, _EMB), lambda i: (0, 0)),
            pl.BlockSpec((1, _EDGE_BLK // _EMB, _EMB), lambda i: (i, 0, 0)),
        ],
        out_shape=[
            jax.ShapeDtypeStruct((_TROWS, _EMB), jnp.float32),
            jax.ShapeDtypeStruct((n // _EDGE_BLK, _EDGE_BLK // _EMB, _EMB), jnp.int32),
        ],
        interpret=interpret,
    )(edge_attr, W0, W1, W2)


def _sc_gather(cidx, table):
    n = cidx.shape[0]
    bpw = n // _NW
    nfull = bpw // _CHUNK
    tail = bpw - nfull * _CHUNK
    mesh = plsc.VectorSubcoreMesh(core_axis_name="c", subcore_axis_name="s")

    @functools.partial(
        pl.kernel,
        out_type=jax.ShapeDtypeStruct((n, _EMB), jnp.float32),
        mesh=mesh,
        scratch_types=[
            pltpu.VMEM((bpw,), jnp.int32),
            pltpu.VMEM((2, _CHUNK, _EMB), jnp.float32),
            pltpu.SemaphoreType.DMA,
            pltpu.SemaphoreType.DMA,
        ],
    )
    def k(cidx_hbm, t_hbm, out_hbm, idx_v, rows_v, gsem, ssem):
        wid = lax.axis_index("s") * _NC + lax.axis_index("c")
        base = wid * bpw
        pltpu.sync_copy(cidx_hbm.at[pl.ds(base, bpw)], idx_v)

        def chunk_pair(it, _):
            j0 = it * 2
            g0 = pltpu.async_copy(
                t_hbm.at[idx_v.at[pl.ds(j0 * _CHUNK, _CHUNK)]], rows_v.at[0], gsem)
            g1 = pltpu.async_copy(
                t_hbm.at[idx_v.at[pl.ds((j0 + 1) * _CHUNK, _CHUNK)]], rows_v.at[1], gsem)
            g0.wait()
            s0 = pltpu.async_copy(
                rows_v.at[0], out_hbm.at[pl.ds(base + j0 * _CHUNK, _CHUNK)], ssem)
            g1.wait()
            s1 = pltpu.async_copy(
                rows_v.at[1], out_hbm.at[pl.ds(base + (j0 + 1) * _CHUNK, _CHUNK)], ssem)
            s0.wait()
            s1.wait()
            return None

        lax.fori_loop(0, nfull // 2, chunk_pair, None)

        if tail:
            gt = pltpu.async_copy(
                t_hbm.at[idx_v.at[pl.ds(nfull * _CHUNK, tail)]],
                rows_v.at[0, pl.ds(0, tail)], gsem)
            gt.wait()
            pltpu.sync_copy(
                rows_v.at[0, pl.ds(0, tail)],
                out_hbm.at[pl.ds(base + nfull * _CHUNK, tail)])

    return k(cidx, table)


def kernel(edge_attr, W0, W1, W2):
    table, cidx = _tc_prep(edge_attr, W0, W1, W2)
    return _sc_gather(cidx.reshape(-1), table)
